# baseline (speedup 1.0000x reference)
import functools

import jax
import jax.numpy as jnp
from jax import lax
from jax.experimental import pallas as pl
from jax.experimental.pallas import tpu as pltpu
from jax.experimental.pallas import tpu_sc as plsc

_LAMBDA_POS = 0.5
_LAMBDA_RAD = 0.5
_LAMBDA_UNPAIR = 0.5
_N = 20
_M = 12
_L = 16

_ROWS = 67
_PX, _PY, _PR = 0, 20, 40
_LX, _LY, _LR = 60, 61, 62
_PPROB, _PRAD = 63, 65

_LN2 = 0.6931471805599453


def _vsqrt(x):
    bits = lax.bitcast_convert_type(x, jnp.int32)
    y = lax.bitcast_convert_type(jnp.int32(0x5F3759DF) - (bits >> 1), jnp.float32)
    for _ in range(3):
        y = y * (1.5 - 0.5 * x * y * y)
    return x * y


def _vlog(x):
    bits = lax.bitcast_convert_type(x, jnp.int32)
    e = ((bits >> 23) & 0xFF) - 127
    m = lax.bitcast_convert_type((bits & 0x007FFFFF) | 0x3F800000, jnp.float32)
    big = m > 1.4142135623730951
    m = jnp.where(big, m * 0.5, m)
    e = jnp.where(big, e + 1, e)
    t = (m - 1.0) / (m + 1.0)
    t2 = t * t
    s = 2.0 * t * (1.0 + t2 * (1.0 / 3.0 + t2 * (0.2 + t2 * (1.0 / 7.0 + t2 * (1.0 / 9.0)))))
    return e.astype(jnp.float32) * _LN2 + s


def _body(inp_hbm, out_hbm, inp_v, mask_v, ppcol_v, out_v):
    cid = lax.axis_index("c")
    sid = lax.axis_index("s")

    @pl.when(jnp.logical_and(cid == 0, sid == 0))
    def _():
        pltpu.sync_copy(inp_hbm, inp_v)

        lane = lax.iota(jnp.int32, _L)
        zeros = jnp.zeros((_L,), jnp.float32)
        mask_v[0:_L] = zeros
        mask_v[_L:2 * _L] = zeros

        lx = inp_v[_LX]
        ly = inp_v[_LY]
        lr = inp_v[_LR]

        mincost = jnp.full((_L,), 3.0e38, jnp.float32)
        amin = jnp.zeros((_L,), jnp.int32)
        for n in range(_N):
            dx = lx - inp_v[_PX + n]
            dy = ly - inp_v[_PY + n]
            dist = _vsqrt(dx * dx + dy * dy)
            rdiff = jnp.abs(lr - inp_v[_PR + n])
            cost = _LAMBDA_POS * dist + _LAMBDA_RAD * rdiff
            upd = cost < mincost
            mincost = jnp.where(upd, cost, mincost)
            amin = jnp.where(upd, n, amin)

        label_ok = lane < _M
        ppcol_v[0:_L] = inp_v[_PPROB]
        ppcol_v[_L:2 * _L] = inp_v[_PPROB + 1]
        sel_p = plsc.load_gather(ppcol_v, [amin])
        pair = mincost + (-_vlog(sel_p + 1.0e-6))
        loss_pair = jnp.sum(jnp.where(label_ok, pair, 0.0))

        plsc.store_scatter(mask_v, [amin], jnp.ones((_L,), jnp.float32),
                           mask=label_ok)

        loss_unpair = jnp.float32(0.0)
        for half in range(2):
            pm = mask_v[pl.ds(half * _L, _L)]
            pp = inp_v[_PPROB + half]
            pr = inp_v[_PRAD + half]
            term = (-_vlog(1.0 - pp + 1.0e-6) + _LAMBDA_RAD * pr) * _LAMBDA_UNPAIR
            ok = jnp.logical_and(pm == 0.0, (lane + half * _L) < _N)
            loss_unpair = loss_unpair + jnp.sum(jnp.where(ok, term, 0.0))

        loss = loss_pair * (1.0 / _M) + loss_unpair * (1.0 / (_N - _M))
        out_v[:] = jnp.broadcast_to(loss, (_L,))
        pltpu.sync_copy(out_v, out_hbm)


_sc_loss = pl.kernel(
    _body,
    out_type=jax.ShapeDtypeStruct((_L,), jnp.float32),
    mesh=plsc.VectorSubcoreMesh(core_axis_name="c", subcore_axis_name="s"),
    compiler_params=pltpu.CompilerParams(needs_layout_passes=False),
    scratch_types=[
        pltpu.VMEM((_ROWS, _L), jnp.float32),
        pltpu.VMEM((2 * _L,), jnp.float32),
        pltpu.VMEM((2 * _L,), jnp.float32),
        pltpu.VMEM((_L,), jnp.float32),
    ],
)


@jax.jit
def kernel(pred, label):
    predb = jnp.broadcast_to(
        pred.T[:3, :, None], (3, _N, _L)).reshape(3 * _N, _L)
    lab = jnp.zeros((3, _L), jnp.float32).at[:, :_M].set(label.T[:3])
    pp = jnp.zeros((2 * _L,), jnp.float32).at[:_N].set(pred[:, 3]).reshape(2, _L)
    pr = jnp.zeros((2 * _L,), jnp.float32).at[:_N].set(pred[:, 2]).reshape(2, _L)
    inp = jnp.concatenate([predb, lab, pp, pr], axis=0)
    return _sc_loss(inp)[0]


# speedup vs baseline: 1.0997x; 1.0997x over previous
import jax
import jax.numpy as jnp
from jax import lax
from jax.experimental import pallas as pl
from jax.experimental.pallas import tpu as pltpu
from jax.experimental.pallas import tpu_sc as plsc

_LAMBDA_POS = 0.5
_LAMBDA_RAD = 0.5
_LAMBDA_UNPAIR = 0.5
_N = 20
_M = 12
_L = 16

_LN2 = 0.6931471805599453


def _vsqrt(x):
    bits = lax.bitcast_convert_type(x, jnp.int32)
    y = lax.bitcast_convert_type(jnp.int32(0x5F3759DF) - (bits >> 1), jnp.float32)
    for _ in range(3):
        y = y * (1.5 - 0.5 * x * y * y)
    return x * y


def _vlog(x):
    bits = lax.bitcast_convert_type(x, jnp.int32)
    e = ((bits >> 23) & 0xFF) - 127
    m = lax.bitcast_convert_type((bits & 0x007FFFFF) | 0x3F800000, jnp.float32)
    big = m > 1.4142135623730951
    m = jnp.where(big, m * 0.5, m)
    e = jnp.where(big, e + 1, e)
    t = (m - 1.0) / (m + 1.0)
    t2 = t * t
    s = 2.0 * t * (1.0 + t2 * (1.0 / 3.0 + t2 * (0.2 + t2 * (1.0 / 7.0 + t2 * (1.0 / 9.0)))))
    return e.astype(jnp.float32) * _LN2 + s


def _body(pred_hbm, label_hbm, out_hbm, pv, lv, mask_v, out_v, sem0, sem1):
    if True:
        c1 = pltpu.async_copy(pred_hbm, pv, sem0)
        c2 = pltpu.async_copy(label_hbm, lv, sem1)
        c1.wait()
        c2.wait()

        lane = lax.iota(jnp.int32, _L)
        zeros = jnp.zeros((_L,), jnp.float32)
        mask_v[0:_L] = zeros
        mask_v[_L:2 * _L] = zeros

        lidx = jnp.minimum(lane, _M - 1) * 4
        lx = plsc.load_gather(lv, [lidx])
        ly = plsc.load_gather(lv, [lidx + 1])
        lr = plsc.load_gather(lv, [lidx + 2])

        mincost = jnp.full((_L,), 3.0e38, jnp.float32)
        amin = jnp.zeros((_L,), jnp.int32)
        for n in range(_N):
            px = plsc.load_gather(pv, [jnp.full((_L,), 4 * n, jnp.int32)])
            py = plsc.load_gather(pv, [jnp.full((_L,), 4 * n + 1, jnp.int32)])
            pr = plsc.load_gather(pv, [jnp.full((_L,), 4 * n + 2, jnp.int32)])
            dx = lx - px
            dy = ly - py
            dist = _vsqrt(dx * dx + dy * dy)
            rdiff = jnp.abs(lr - pr)
            cost = _LAMBDA_POS * dist + _LAMBDA_RAD * rdiff
            upd = cost < mincost
            mincost = jnp.where(upd, cost, mincost)
            amin = jnp.where(upd, n, amin)

        label_ok = lane < _M
        sel_p = plsc.load_gather(pv, [amin * 4 + 3])
        pair = mincost + (-_vlog(sel_p + 1.0e-6))
        loss_pair = jnp.sum(jnp.where(label_ok, pair, 0.0))

        plsc.store_scatter(mask_v, [amin], jnp.ones((_L,), jnp.float32),
                           mask=label_ok)

        loss_unpair = jnp.float32(0.0)
        for half in range(2):
            pm = mask_v[pl.ds(half * _L, _L)]
            pidx = jnp.minimum(lane + half * _L, _N - 1)
            pp = plsc.load_gather(pv, [pidx * 4 + 3])
            pr = plsc.load_gather(pv, [pidx * 4 + 2])
            term = (-_vlog(1.0 - pp + 1.0e-6) + _LAMBDA_RAD * pr) * _LAMBDA_UNPAIR
            ok = jnp.logical_and(pm == 0.0, (lane + half * _L) < _N)
            loss_unpair = loss_unpair + jnp.sum(jnp.where(ok, term, 0.0))

        loss = loss_pair * (1.0 / _M) + loss_unpair * (1.0 / (_N - _M))
        out_v[:] = jnp.broadcast_to(loss, (_L,))
        pltpu.sync_copy(out_v, out_hbm)


_sc_loss = pl.kernel(
    _body,
    out_type=jax.ShapeDtypeStruct((_L,), jnp.float32),
    mesh=plsc.VectorSubcoreMesh(core_axis_name="c", subcore_axis_name="s",
                                num_cores=1, num_subcores=1),
    compiler_params=pltpu.CompilerParams(
        needs_layout_passes=False,
        disable_bounds_checks=True,
        disable_semaphore_checks=True,
        skip_device_barrier=True,
    ),
    scratch_types=[
        pltpu.VMEM((4 * _N,), jnp.float32),
        pltpu.VMEM((4 * _M,), jnp.float32),
        pltpu.VMEM((2 * _L,), jnp.float32),
        pltpu.VMEM((_L,), jnp.float32),
        pltpu.SemaphoreType.DMA,
        pltpu.SemaphoreType.DMA,
    ],
)


@jax.jit
def kernel(pred, label):
    return _sc_loss(pred.reshape(4 * _N), label.reshape(4 * _M))[0]


# speedup vs baseline: 1.1259x; 1.0238x over previous
import jax
import jax.numpy as jnp
from jax import lax
from jax.experimental import pallas as pl
from jax.experimental.pallas import tpu as pltpu
from jax.experimental.pallas import tpu_sc as plsc

_LAMBDA_POS = 0.5
_LAMBDA_RAD = 0.5
_LAMBDA_UNPAIR = 0.5
_N = 20
_M = 12
_L = 16

_LN2 = 0.6931471805599453


def _vsqrt(x):
    bits = lax.bitcast_convert_type(x, jnp.int32)
    y = lax.bitcast_convert_type(jnp.int32(0x5F3759DF) - (bits >> 1), jnp.float32)
    for _ in range(3):
        y = y * (1.5 - 0.5 * x * y * y)
    return x * y


def _vlog(x):
    bits = lax.bitcast_convert_type(x, jnp.int32)
    e = ((bits >> 23) & 0xFF) - 127
    m = lax.bitcast_convert_type((bits & 0x007FFFFF) | 0x3F800000, jnp.float32)
    big = m > 1.4142135623730951
    m = jnp.where(big, m * 0.5, m)
    e = jnp.where(big, e + 1, e)
    t = (m - 1.0) / (m + 1.0)
    t2 = t * t
    s = 2.0 * t * (1.0 + t2 * (1.0 / 3.0 + t2 * (0.2 + t2 * (1.0 / 7.0 + t2 * (1.0 / 9.0)))))
    return e.astype(jnp.float32) * _LN2 + s


def _body(pred_hbm, label_hbm, out_hbm, pv, lv, mask_v, out_v, sem0, sem1):
    if True:
        c1 = pltpu.async_copy(pred_hbm, pv, sem0)
        c2 = pltpu.async_copy(label_hbm, lv, sem1)
        c1.wait()
        c2.wait()

        lane = lax.iota(jnp.int32, _L)
        zeros = jnp.zeros((_L,), jnp.float32)
        mask_v[0:_L] = zeros
        mask_v[_L:2 * _L] = zeros

        lidx = jnp.minimum(lane, _M - 1) * 4
        lx = plsc.load_gather(lv, [lidx])
        ly = plsc.load_gather(lv, [lidx + 1])
        lr = plsc.load_gather(lv, [lidx + 2])

        def _step(n, carry):
            mincost, amin = carry
            base = jnp.broadcast_to(4 * n, (_L,)).astype(jnp.int32)
            px = plsc.load_gather(pv, [base])
            py = plsc.load_gather(pv, [base + 1])
            pr = plsc.load_gather(pv, [base + 2])
            dx = lx - px
            dy = ly - py
            dist = _vsqrt(dx * dx + dy * dy)
            rdiff = jnp.abs(lr - pr)
            cost = _LAMBDA_POS * dist + _LAMBDA_RAD * rdiff
            upd = cost < mincost
            return (jnp.where(upd, cost, mincost), jnp.where(upd, n, amin))

        mincost, amin = lax.fori_loop(
            0, _N,
            _step,
            (jnp.full((_L,), 3.0e38, jnp.float32), jnp.zeros((_L,), jnp.int32)),
        )

        label_ok = lane < _M
        sel_p = plsc.load_gather(pv, [amin * 4 + 3])
        pair = mincost + (-_vlog(sel_p + 1.0e-6))
        loss_pair = jnp.sum(jnp.where(label_ok, pair, 0.0))

        plsc.store_scatter(mask_v, [amin], jnp.ones((_L,), jnp.float32),
                           mask=label_ok)

        loss_unpair = jnp.float32(0.0)
        for half in range(2):
            pm = mask_v[pl.ds(half * _L, _L)]
            pidx = jnp.minimum(lane + half * _L, _N - 1)
            pp = plsc.load_gather(pv, [pidx * 4 + 3])
            pr = plsc.load_gather(pv, [pidx * 4 + 2])
            term = (-_vlog(1.0 - pp + 1.0e-6) + _LAMBDA_RAD * pr) * _LAMBDA_UNPAIR
            ok = jnp.logical_and(pm == 0.0, (lane + half * _L) < _N)
            loss_unpair = loss_unpair + jnp.sum(jnp.where(ok, term, 0.0))

        loss = loss_pair * (1.0 / _M) + loss_unpair * (1.0 / (_N - _M))
        out_v[:] = jnp.broadcast_to(loss, (_L,))
        pltpu.sync_copy(out_v, out_hbm)


_sc_loss = pl.kernel(
    _body,
    out_type=jax.ShapeDtypeStruct((_L,), jnp.float32),
    mesh=plsc.VectorSubcoreMesh(core_axis_name="c", subcore_axis_name="s",
                                num_cores=1, num_subcores=1),
    compiler_params=pltpu.CompilerParams(
        needs_layout_passes=False,
        disable_bounds_checks=True,
        disable_semaphore_checks=True,
        skip_device_barrier=True,
    ),
    scratch_types=[
        pltpu.VMEM((4 * _N,), jnp.float32),
        pltpu.VMEM((4 * _M,), jnp.float32),
        pltpu.VMEM((2 * _L,), jnp.float32),
        pltpu.VMEM((_L,), jnp.float32),
        pltpu.SemaphoreType.DMA,
        pltpu.SemaphoreType.DMA,
    ],
)


@jax.jit
def kernel(pred, label):
    return _sc_loss(pred.reshape(4 * _N), label.reshape(4 * _M))[0]
